# baseline (speedup 1.0000x reference)
import jax
import jax.numpy as jnp
from jax import lax
from jax.experimental import pallas as pl
from jax.experimental.pallas import tpu as pltpu
from jax.experimental.pallas import tpu_sc as plsc

N = 10000
E = 320000
D = 128

NC = 2
NS = 16
NW = NC * NS
NPAD = 10240
ROWS_PER_TILE = NPAD // NS
CHUNK = 128
G = 40
HEAVY_CORE = 1
C_HEAVY = 160
C_LIGHT = 0
TOT_CHUNKS = NS * (C_HEAVY + C_LIGHT)
EPAD = TOT_CHUNKS * CHUNK
NBLK = TOT_CHUNKS // G


def _sc_body(x_hbm, src_hbm, dst_hbm, zrow_hbm,
             agg_out, cnt_out,
             src_i, dst_i, rows0, rows1, ones_v, cnt_v,
             agg_sh, cnt_sh, sem0, sem1):
    c = lax.axis_index("c")
    s = lax.axis_index("s")
    row0 = s * ROWS_PER_TILE

    pltpu.sync_copy(zrow_hbm, agg_sh.at[pl.ds(row0, ROWS_PER_TILE)])
    z16 = jnp.zeros((16,), jnp.float32)
    for i in range(ROWS_PER_TILE // 16):
        cnt_v[pl.ds(i * 16, 16)] = z16
    pltpu.sync_copy(cnt_v, cnt_sh.at[pl.ds(row0, ROWS_PER_TILE)])
    o16 = jnp.ones((16,), jnp.float32)
    for i in range(CHUNK // 16):
        ones_v[pl.ds(i * 16, 16)] = o16
    plsc.subcore_barrier()

    rows = (rows0, rows1)
    sems = (sem0, sem1)
    nstage = jnp.where(c == HEAVY_CORE, C_HEAVY // G, C_LIGHT // G)
    blk0 = jnp.where(c == HEAVY_CORE, s * (C_HEAVY // G),
                     NS * (C_HEAVY // G) + s * (C_LIGHT // G))

    def gather(j, b):
        return pltpu.async_copy(x_hbm.at[src_i.at[j]], rows[b], sems[b])

    def scatter(j, b):
        pltpu.make_async_copy(x_hbm.at[src_i.at[j]], rows[b],
                              sems[b]).wait()
        pltpu.sync_copy(rows[b], agg_sh.at[dst_i.at[j]], add=True)
        pltpu.sync_copy(ones_v, cnt_sh.at[dst_i.at[j]], add=True)

    def stage(st, carry):
        pltpu.sync_copy(src_hbm.at[blk0 + st], src_i)
        pltpu.sync_copy(dst_hbm.at[blk0 + st], dst_i)
        gather(0, 0)

        def pair(i, carry2):
            j0 = 2 * i
            gather(j0 + 1, 1)
            scatter(j0, 0)

            @pl.when(i < G // 2 - 1)
            def _():
                gather(j0 + 2, 0)

            scatter(j0 + 1, 1)
            return carry2

        lax.fori_loop(0, G // 2, pair, 0)
        return carry

    lax.fori_loop(0, nstage, stage, 0)
    plsc.subcore_barrier()

    pltpu.sync_copy(agg_sh.at[pl.ds(row0, ROWS_PER_TILE)],
                    agg_out.at[c, pl.ds(row0, ROWS_PER_TILE)])
    pltpu.sync_copy(cnt_sh.at[pl.ds(row0, ROWS_PER_TILE)],
                    cnt_out.at[c, pl.ds(row0, ROWS_PER_TILE)])


@jax.jit
def _sc_aggregate(x, src, dst, zrow):
    mesh = plsc.VectorSubcoreMesh(core_axis_name="c", subcore_axis_name="s",
                                  num_cores=NC, num_subcores=NS)
    return pl.kernel(
        _sc_body,
        out_type=[
            jax.ShapeDtypeStruct((NC, NPAD, D), jnp.float32),
            jax.ShapeDtypeStruct((NC, NPAD), jnp.float32),
        ],
        mesh=mesh,
        scratch_types=[
            pltpu.VMEM((G, CHUNK), jnp.int32),
            pltpu.VMEM((G, CHUNK), jnp.int32),
            pltpu.VMEM((CHUNK, D), jnp.float32),
            pltpu.VMEM((CHUNK, D), jnp.float32),
            pltpu.VMEM((CHUNK,), jnp.float32),
            pltpu.VMEM((ROWS_PER_TILE,), jnp.float32),
            pltpu.VMEM_SHARED((NPAD, D), jnp.float32),
            pltpu.VMEM_SHARED((NPAD,), jnp.float32),
            pltpu.SemaphoreType.DMA,
            pltpu.SemaphoreType.DMA,
        ],
    )(x, src, dst, zrow)


BLK = 1024


def _tc_body(scal_ref, agg0, agg1, cnt0, cnt1, x_ref, rr_ref,
             wlT_ref, wrT_ref, brow_ref, wsT_ref, out_ref):
    a = scal_ref[0]
    bs_ = scal_ref[1]
    agg = agg0[...] + agg1[...]
    cnt = jnp.maximum(cnt0[...] + cnt1[...], 1.0)
    mean = agg / cnt
    xb = x_ref[...]
    z = (jnp.dot(mean, wlT_ref[...], preferred_element_type=jnp.float32)
         + jnp.dot(xb, wrT_ref[...], preferred_element_type=jnp.float32)
         + brow_ref[...])
    h = jnp.maximum(z, 0.0) + xb
    gnn = jnp.dot(h, wsT_ref[...], preferred_element_type=jnp.float32) + bs_
    out_ref[...] = a * rr_ref[...] + (1.0 - a) * gnn


@jax.jit
def _tc_tail(scal, agg0, agg1, cnt0, cnt1, xp, rrp, wlT, wrT, brow, wsT):
    grid = (NPAD // BLK,)
    return pl.pallas_call(
        _tc_body,
        grid=grid,
        in_specs=[
            pl.BlockSpec(memory_space=pltpu.SMEM),
            pl.BlockSpec((BLK, D), lambda i: (i, 0)),
            pl.BlockSpec((BLK, D), lambda i: (i, 0)),
            pl.BlockSpec((BLK, 1), lambda i: (i, 0)),
            pl.BlockSpec((BLK, 1), lambda i: (i, 0)),
            pl.BlockSpec((BLK, D), lambda i: (i, 0)),
            pl.BlockSpec((BLK, 1), lambda i: (i, 0)),
            pl.BlockSpec((D, D), lambda i: (0, 0)),
            pl.BlockSpec((D, D), lambda i: (0, 0)),
            pl.BlockSpec((1, D), lambda i: (0, 0)),
            pl.BlockSpec((D, 1), lambda i: (0, 0)),
        ],
        out_specs=pl.BlockSpec((BLK, 1), lambda i: (i, 0)),
        out_shape=jax.ShapeDtypeStruct((NPAD, 1), jnp.float32),
    )(scal, agg0, agg1, cnt0, cnt1, xp, rrp, wlT, wrT, brow, wsT)


def kernel(x, edge_index, reranker_scores, Wl, bl, Wr, br, Ws, bs, alpha):
    src = edge_index[0].astype(jnp.int32)
    dst = edge_index[1].astype(jnp.int32)
    src = jnp.concatenate([src, jnp.zeros((EPAD - E,), jnp.int32)])
    dst = jnp.concatenate([dst, jnp.full((EPAD - E,), N, jnp.int32)])
    src = src.reshape(NBLK, G, CHUNK)
    dst = dst.reshape(NBLK, G, CHUNK)
    zrow = jnp.zeros((ROWS_PER_TILE, D), jnp.float32)

    agg, cnt = _sc_aggregate(x, src, dst, zrow)

    xp = jnp.concatenate([x, jnp.zeros((NPAD - N, D), jnp.float32)])
    rrp = jnp.concatenate([reranker_scores,
                           jnp.zeros((NPAD - N,), jnp.float32)])[:, None]
    a = jax.nn.sigmoid(alpha)
    scal = jnp.stack([a, bs[0]])
    brow = (bl + br)[None, :]
    out = _tc_tail(scal, agg[0], agg[1], cnt[0, :, None], cnt[1, :, None],
                   xp, rrp, Wl.T, Wr.T, brow, Ws.T)
    return out[:N, 0]


# speedup vs baseline: 1.1916x; 1.1916x over previous
import jax
import jax.numpy as jnp
from jax import lax
from jax.experimental import pallas as pl
from jax.experimental.pallas import tpu as pltpu
from jax.experimental.pallas import tpu_sc as plsc

N = 10000
E = 320000
D = 128

NC = 2
NS = 16
NW = NC * NS
NPAD = 10240
ROWS_PER_TILE = NPAD // NS
CHUNK = 128
G = 40
HEAVY_CORE = 1
C_HEAVY = 80
C_LIGHT = 80
TOT_CHUNKS = NS * (C_HEAVY + C_LIGHT)
EPAD = TOT_CHUNKS * CHUNK
NBLK = TOT_CHUNKS // G


def _sc_body(x_hbm, src_hbm, dst_hbm, zrow_hbm,
             agg_out, cnt_out,
             src_i, dst_i, rows0, rows1, ones_v, cnt_v,
             agg_sh, cnt_sh, sem0, sem1):
    c = lax.axis_index("c")
    s = lax.axis_index("s")
    row0 = s * ROWS_PER_TILE

    pltpu.sync_copy(zrow_hbm, agg_sh.at[pl.ds(row0, ROWS_PER_TILE)])
    z16 = jnp.zeros((16,), jnp.float32)
    for i in range(ROWS_PER_TILE // 16):
        cnt_v[pl.ds(i * 16, 16)] = z16
    pltpu.sync_copy(cnt_v, cnt_sh.at[pl.ds(row0, ROWS_PER_TILE)])
    o16 = jnp.ones((16,), jnp.float32)
    for i in range(CHUNK // 16):
        ones_v[pl.ds(i * 16, 16)] = o16
    plsc.subcore_barrier()

    rows = (rows0, rows1)
    sems = (sem0, sem1)
    nstage = jnp.where(c == HEAVY_CORE, C_HEAVY // G, C_LIGHT // G)
    blk0 = jnp.where(c == HEAVY_CORE, s * (C_HEAVY // G),
                     NS * (C_HEAVY // G) + s * (C_LIGHT // G))

    def gather(j, b):
        return pltpu.async_copy(x_hbm.at[src_i.at[j]], rows[b], sems[b])

    def scatter(j, b):
        pltpu.make_async_copy(x_hbm.at[src_i.at[j]], rows[b],
                              sems[b]).wait()
        pltpu.sync_copy(rows[b], agg_sh.at[dst_i.at[j]], add=True)
        pltpu.sync_copy(ones_v, cnt_sh.at[dst_i.at[j]], add=True)

    def stage(st, carry):
        pltpu.sync_copy(src_hbm.at[blk0 + st], src_i)
        pltpu.sync_copy(dst_hbm.at[blk0 + st], dst_i)
        gather(0, 0)

        def pair(i, carry2):
            j0 = 2 * i
            gather(j0 + 1, 1)
            scatter(j0, 0)

            @pl.when(i < G // 2 - 1)
            def _():
                gather(j0 + 2, 0)

            scatter(j0 + 1, 1)
            return carry2

        lax.fori_loop(0, G // 2, pair, 0)
        return carry

    lax.fori_loop(0, nstage, stage, 0)
    plsc.subcore_barrier()

    pltpu.sync_copy(agg_sh.at[pl.ds(row0, ROWS_PER_TILE)],
                    agg_out.at[c, pl.ds(row0, ROWS_PER_TILE)])
    pltpu.sync_copy(cnt_sh.at[pl.ds(row0, ROWS_PER_TILE)],
                    cnt_out.at[c, pl.ds(row0, ROWS_PER_TILE)])


@jax.jit
def _sc_aggregate(x, src, dst, zrow):
    mesh = plsc.VectorSubcoreMesh(core_axis_name="c", subcore_axis_name="s",
                                  num_cores=NC, num_subcores=NS)
    return pl.kernel(
        _sc_body,
        out_type=[
            jax.ShapeDtypeStruct((NC, NPAD, D), jnp.float32),
            jax.ShapeDtypeStruct((NC, NPAD), jnp.float32),
        ],
        mesh=mesh,
        scratch_types=[
            pltpu.VMEM((G, CHUNK), jnp.int32),
            pltpu.VMEM((G, CHUNK), jnp.int32),
            pltpu.VMEM((CHUNK, D), jnp.float32),
            pltpu.VMEM((CHUNK, D), jnp.float32),
            pltpu.VMEM((CHUNK,), jnp.float32),
            pltpu.VMEM((ROWS_PER_TILE,), jnp.float32),
            pltpu.VMEM_SHARED((NPAD, D), jnp.float32),
            pltpu.VMEM_SHARED((NPAD,), jnp.float32),
            pltpu.SemaphoreType.DMA,
            pltpu.SemaphoreType.DMA,
        ],
    )(x, src, dst, zrow)


BLK = 1024


def _tc_body(scal_ref, agg0, agg1, cnt0, cnt1, x_ref, rr_ref,
             wlT_ref, wrT_ref, brow_ref, wsT_ref, out_ref):
    a = scal_ref[0]
    bs_ = scal_ref[1]
    agg = agg0[...] + agg1[...]
    cnt = jnp.maximum(cnt0[...] + cnt1[...], 1.0)
    mean = agg / cnt
    xb = x_ref[...]
    z = (jnp.dot(mean, wlT_ref[...], preferred_element_type=jnp.float32)
         + jnp.dot(xb, wrT_ref[...], preferred_element_type=jnp.float32)
         + brow_ref[...])
    h = jnp.maximum(z, 0.0) + xb
    gnn = jnp.dot(h, wsT_ref[...], preferred_element_type=jnp.float32) + bs_
    out_ref[...] = a * rr_ref[...] + (1.0 - a) * gnn


@jax.jit
def _tc_tail(scal, agg0, agg1, cnt0, cnt1, xp, rrp, wlT, wrT, brow, wsT):
    grid = (NPAD // BLK,)
    return pl.pallas_call(
        _tc_body,
        grid=grid,
        in_specs=[
            pl.BlockSpec(memory_space=pltpu.SMEM),
            pl.BlockSpec((BLK, D), lambda i: (i, 0)),
            pl.BlockSpec((BLK, D), lambda i: (i, 0)),
            pl.BlockSpec((BLK, 1), lambda i: (i, 0)),
            pl.BlockSpec((BLK, 1), lambda i: (i, 0)),
            pl.BlockSpec((BLK, D), lambda i: (i, 0)),
            pl.BlockSpec((BLK, 1), lambda i: (i, 0)),
            pl.BlockSpec((D, D), lambda i: (0, 0)),
            pl.BlockSpec((D, D), lambda i: (0, 0)),
            pl.BlockSpec((1, D), lambda i: (0, 0)),
            pl.BlockSpec((D, 1), lambda i: (0, 0)),
        ],
        out_specs=pl.BlockSpec((BLK, 1), lambda i: (i, 0)),
        out_shape=jax.ShapeDtypeStruct((NPAD, 1), jnp.float32),
    )(scal, agg0, agg1, cnt0, cnt1, xp, rrp, wlT, wrT, brow, wsT)


def kernel(x, edge_index, reranker_scores, Wl, bl, Wr, br, Ws, bs, alpha):
    src = edge_index[0].astype(jnp.int32)
    dst = edge_index[1].astype(jnp.int32)
    src = jnp.concatenate([src, jnp.zeros((EPAD - E,), jnp.int32)])
    dst = jnp.concatenate([dst, jnp.full((EPAD - E,), N, jnp.int32)])
    src = src.reshape(NBLK, G, CHUNK)
    dst = dst.reshape(NBLK, G, CHUNK)
    zrow = jnp.zeros((ROWS_PER_TILE, D), jnp.float32)

    agg, cnt = _sc_aggregate(x, src, dst, zrow)

    xp = jnp.concatenate([x, jnp.zeros((NPAD - N, D), jnp.float32)])
    rrp = jnp.concatenate([reranker_scores,
                           jnp.zeros((NPAD - N,), jnp.float32)])[:, None]
    a = jax.nn.sigmoid(alpha)
    scal = jnp.stack([a, bs[0]])
    brow = (bl + br)[None, :]
    out = _tc_tail(scal, agg[0], agg[1], cnt[0, :, None], cnt[1, :, None],
                   xp, rrp, Wl.T, Wr.T, brow, Ws.T)
    return out[:N, 0]


# speedup vs baseline: 1.2699x; 1.0657x over previous
import jax
import jax.numpy as jnp
from jax import lax
from jax.experimental import pallas as pl
from jax.experimental.pallas import tpu as pltpu
from jax.experimental.pallas import tpu_sc as plsc

N = 10000
E = 320000
D = 128

NC = 2
NS = 16
NW = NC * NS
NPAD = 10240
ROWS_PER_TILE = NPAD // NS
CHUNK = 128
G = 40
HEAVY_CORE = 1
C_HEAVY = 120
C_LIGHT = 40
TOT_CHUNKS = NS * (C_HEAVY + C_LIGHT)
EPAD = TOT_CHUNKS * CHUNK
NBLK = TOT_CHUNKS // G


def _sc_body(x_hbm, src_hbm, dst_hbm, zrow_hbm,
             agg_out, cnt_out,
             src_i, dst_i, rows0, rows1, ones_v, cnt_v,
             agg_sh, cnt_sh, sem0, sem1):
    c = lax.axis_index("c")
    s = lax.axis_index("s")
    row0 = s * ROWS_PER_TILE

    pltpu.sync_copy(zrow_hbm, agg_sh.at[pl.ds(row0, ROWS_PER_TILE)])
    z16 = jnp.zeros((16,), jnp.float32)
    for i in range(ROWS_PER_TILE // 16):
        cnt_v[pl.ds(i * 16, 16)] = z16
    pltpu.sync_copy(cnt_v, cnt_sh.at[pl.ds(row0, ROWS_PER_TILE)])
    o16 = jnp.ones((16,), jnp.float32)
    for i in range(CHUNK // 16):
        ones_v[pl.ds(i * 16, 16)] = o16
    plsc.subcore_barrier()

    rows = (rows0, rows1)
    sems = (sem0, sem1)
    nstage = jnp.where(c == HEAVY_CORE, C_HEAVY // G, C_LIGHT // G)
    blk0 = jnp.where(c == HEAVY_CORE, s * (C_HEAVY // G),
                     NS * (C_HEAVY // G) + s * (C_LIGHT // G))

    def gather(j, b):
        return pltpu.async_copy(x_hbm.at[src_i.at[j]], rows[b], sems[b])

    def scatter(j, b):
        pltpu.make_async_copy(x_hbm.at[src_i.at[j]], rows[b],
                              sems[b]).wait()
        pltpu.sync_copy(rows[b], agg_sh.at[dst_i.at[j]], add=True)
        pltpu.sync_copy(ones_v, cnt_sh.at[dst_i.at[j]], add=True)

    def stage(st, carry):
        pltpu.sync_copy(src_hbm.at[blk0 + st], src_i)
        pltpu.sync_copy(dst_hbm.at[blk0 + st], dst_i)
        gather(0, 0)

        def pair(i, carry2):
            j0 = 2 * i
            gather(j0 + 1, 1)
            scatter(j0, 0)

            @pl.when(i < G // 2 - 1)
            def _():
                gather(j0 + 2, 0)

            scatter(j0 + 1, 1)
            return carry2

        lax.fori_loop(0, G // 2, pair, 0)
        return carry

    lax.fori_loop(0, nstage, stage, 0)
    plsc.subcore_barrier()

    pltpu.sync_copy(agg_sh.at[pl.ds(row0, ROWS_PER_TILE)],
                    agg_out.at[c, pl.ds(row0, ROWS_PER_TILE)])
    pltpu.sync_copy(cnt_sh.at[pl.ds(row0, ROWS_PER_TILE)],
                    cnt_out.at[c, pl.ds(row0, ROWS_PER_TILE)])


@jax.jit
def _sc_aggregate(x, src, dst, zrow):
    mesh = plsc.VectorSubcoreMesh(core_axis_name="c", subcore_axis_name="s",
                                  num_cores=NC, num_subcores=NS)
    return pl.kernel(
        _sc_body,
        out_type=[
            jax.ShapeDtypeStruct((NC, NPAD, D), jnp.float32),
            jax.ShapeDtypeStruct((NC, NPAD), jnp.float32),
        ],
        mesh=mesh,
        scratch_types=[
            pltpu.VMEM((G, CHUNK), jnp.int32),
            pltpu.VMEM((G, CHUNK), jnp.int32),
            pltpu.VMEM((CHUNK, D), jnp.float32),
            pltpu.VMEM((CHUNK, D), jnp.float32),
            pltpu.VMEM((CHUNK,), jnp.float32),
            pltpu.VMEM((ROWS_PER_TILE,), jnp.float32),
            pltpu.VMEM_SHARED((NPAD, D), jnp.float32),
            pltpu.VMEM_SHARED((NPAD,), jnp.float32),
            pltpu.SemaphoreType.DMA,
            pltpu.SemaphoreType.DMA,
        ],
    )(x, src, dst, zrow)


BLK = 1024


def _tc_body(scal_ref, agg0, agg1, cnt0, cnt1, x_ref, rr_ref,
             wlT_ref, wrT_ref, brow_ref, wsT_ref, out_ref):
    a = scal_ref[0]
    bs_ = scal_ref[1]
    agg = agg0[...] + agg1[...]
    cnt = jnp.maximum(cnt0[...] + cnt1[...], 1.0)
    mean = agg / cnt
    xb = x_ref[...]
    z = (jnp.dot(mean, wlT_ref[...], preferred_element_type=jnp.float32)
         + jnp.dot(xb, wrT_ref[...], preferred_element_type=jnp.float32)
         + brow_ref[...])
    h = jnp.maximum(z, 0.0) + xb
    gnn = jnp.dot(h, wsT_ref[...], preferred_element_type=jnp.float32) + bs_
    out_ref[...] = a * rr_ref[...] + (1.0 - a) * gnn


@jax.jit
def _tc_tail(scal, agg0, agg1, cnt0, cnt1, xp, rrp, wlT, wrT, brow, wsT):
    grid = (NPAD // BLK,)
    return pl.pallas_call(
        _tc_body,
        grid=grid,
        in_specs=[
            pl.BlockSpec(memory_space=pltpu.SMEM),
            pl.BlockSpec((BLK, D), lambda i: (i, 0)),
            pl.BlockSpec((BLK, D), lambda i: (i, 0)),
            pl.BlockSpec((BLK, 1), lambda i: (i, 0)),
            pl.BlockSpec((BLK, 1), lambda i: (i, 0)),
            pl.BlockSpec((BLK, D), lambda i: (i, 0)),
            pl.BlockSpec((BLK, 1), lambda i: (i, 0)),
            pl.BlockSpec((D, D), lambda i: (0, 0)),
            pl.BlockSpec((D, D), lambda i: (0, 0)),
            pl.BlockSpec((1, D), lambda i: (0, 0)),
            pl.BlockSpec((D, 1), lambda i: (0, 0)),
        ],
        out_specs=pl.BlockSpec((BLK, 1), lambda i: (i, 0)),
        out_shape=jax.ShapeDtypeStruct((NPAD, 1), jnp.float32),
    )(scal, agg0, agg1, cnt0, cnt1, xp, rrp, wlT, wrT, brow, wsT)


def kernel(x, edge_index, reranker_scores, Wl, bl, Wr, br, Ws, bs, alpha):
    src = edge_index[0].astype(jnp.int32)
    dst = edge_index[1].astype(jnp.int32)
    src = jnp.concatenate([src, jnp.zeros((EPAD - E,), jnp.int32)])
    dst = jnp.concatenate([dst, jnp.full((EPAD - E,), N, jnp.int32)])
    src = src.reshape(NBLK, G, CHUNK)
    dst = dst.reshape(NBLK, G, CHUNK)
    zrow = jnp.zeros((ROWS_PER_TILE, D), jnp.float32)

    agg, cnt = _sc_aggregate(x, src, dst, zrow)

    xp = jnp.concatenate([x, jnp.zeros((NPAD - N, D), jnp.float32)])
    rrp = jnp.concatenate([reranker_scores,
                           jnp.zeros((NPAD - N,), jnp.float32)])[:, None]
    a = jax.nn.sigmoid(alpha)
    scal = jnp.stack([a, bs[0]])
    brow = (bl + br)[None, :]
    out = _tc_tail(scal, agg[0], agg[1], cnt[0, :, None], cnt[1, :, None],
                   xp, rrp, Wl.T, Wr.T, brow, Ws.T)
    return out[:N, 0]
